# baseline (speedup 1.0000x reference)
import functools

import jax
import jax.numpy as jnp
from jax import lax
from jax.experimental import pallas as pl
from jax.experimental.pallas import tpu as pltpu
from jax.experimental.pallas import tpu_sc as plsc

N = 100000
ATOM_ID_DIM = 32
CHARGE_DIM = 8
SHAPE_ID_DIM = 16
MULT_DIM = 16
MOTIF_FEAT_SIZE = 48
MOTIF_DIM = 32
NUM_JOINS = 3
MOTIF_TOT = NUM_JOINS * MOTIF_DIM
OUT_DIM = ATOM_ID_DIM + CHARGE_DIM + MOTIF_TOT + NUM_JOINS * (SHAPE_ID_DIM + MULT_DIM)

COL_CHARGE = ATOM_ID_DIM
COL_MOTIF = COL_CHARGE + CHARGE_DIM
COL_SHAPE = COL_MOTIF + MOTIF_TOT
COL_MULT = COL_SHAPE + NUM_JOINS * SHAPE_ID_DIM


BR = 1000


def _tc_motif_body(motif_ref, w3_ref, b3_ref, out_ref):
    out_ref[...] = jnp.dot(motif_ref[...], w3_ref[...],
                           preferred_element_type=jnp.float32,
                           precision=lax.Precision.HIGHEST) + b3_ref[...]


def _tc_motif(motif, w3, b3):
    return pl.pallas_call(
        _tc_motif_body,
        grid=(N // BR,),
        in_specs=[
            pl.BlockSpec((BR, NUM_JOINS * MOTIF_FEAT_SIZE), lambda i: (i, 0)),
            pl.BlockSpec((NUM_JOINS * MOTIF_FEAT_SIZE, MOTIF_TOT), lambda i: (0, 0)),
            pl.BlockSpec((1, MOTIF_TOT), lambda i: (0, 0)),
        ],
        out_specs=pl.BlockSpec((BR, MOTIF_TOT), lambda i: (i, 0)),
        out_shape=jax.ShapeDtypeStruct((N, MOTIF_TOT), jnp.float32),
        compiler_params=pltpu.CompilerParams(
            dimension_semantics=("arbitrary",),
        ),
    )(motif, w3, b3)


CHUNK = 1000
NUM_CHUNKS = N // CHUNK
CHALF = CHUNK // 2


def _sc_assemble(aidx, cidx, sidx, midx, atom_tab, ctab, stab, mtab, cmat):
    info = plsc.get_sparse_core_info()
    nc, ns = info.num_cores, info.num_subcores
    nw = nc * ns
    chunks_per_w = -(-NUM_CHUNKS // nw)
    mesh = plsc.VectorSubcoreMesh(core_axis_name="c", subcore_axis_name="s")

    @functools.partial(
        pl.kernel,
        mesh=mesh,
        out_type=jax.ShapeDtypeStruct((N, OUT_DIM), jnp.float32),
        scratch_types=[
            pltpu.VMEM((CHUNK,), jnp.int32),
            pltpu.VMEM((CHUNK,), jnp.int32),
            pltpu.VMEM((CHUNK, ATOM_ID_DIM), jnp.float32),
            pltpu.VMEM((CHUNK, CHARGE_DIM), jnp.float32),
            pltpu.VMEM((CHUNK, SHAPE_ID_DIM), jnp.float32),
            pltpu.VMEM((CHALF, MOTIF_TOT), jnp.float32),
            pltpu.SemaphoreType.DMA,
        ],
        compiler_params=pltpu.CompilerParams(use_tc_tiling_on_sc=False),
    )
    def k(aidx_hbm, cidx_hbm, sidx_hbm, midx_hbm, atab_hbm, ctab_hbm, stab_hbm,
          mtab_hbm, cmat_hbm, out, aidx_v, idx_v, arows_v, crows_v, srows_v,
          cbuf_v, sem):
        wid = lax.axis_index("s") * nc + lax.axis_index("c")
        for c in range(chunks_per_w):
            cid = wid + nw * c

            @pl.when(cid < NUM_CHUNKS)
            def _():
                base = cid * CHUNK
                rows = pl.ds(base, CHUNK)
                pltpu.sync_copy(aidx_hbm.at[rows], aidx_v)
                pltpu.async_copy(atab_hbm.at[aidx_v], arows_v, sem).wait()
                pltpu.sync_copy(arows_v, out.at[rows, pl.ds(0, ATOM_ID_DIM)])
                pltpu.sync_copy(cidx_hbm.at[rows], idx_v)
                pltpu.async_copy(ctab_hbm.at[idx_v], crows_v, sem).wait()
                pltpu.sync_copy(crows_v, out.at[rows, pl.ds(COL_CHARGE, CHARGE_DIM)])
                for h in range(CHUNK // CHALF):
                    hrows = pl.ds(base + h * CHALF, CHALF)
                    pltpu.sync_copy(cmat_hbm.at[hrows], cbuf_v)
                    pltpu.sync_copy(cbuf_v, out.at[hrows, pl.ds(COL_MOTIF, MOTIF_TOT)])
                for j in range(NUM_JOINS):
                    pltpu.sync_copy(sidx_hbm.at[pl.ds(j * N + base, CHUNK)], idx_v)
                    pltpu.async_copy(stab_hbm.at[idx_v], srows_v, sem).wait()
                    pltpu.sync_copy(
                        srows_v,
                        out.at[rows, pl.ds(COL_SHAPE + j * SHAPE_ID_DIM, SHAPE_ID_DIM)])
                for j in range(NUM_JOINS):
                    pltpu.sync_copy(midx_hbm.at[pl.ds(j * N + base, CHUNK)], idx_v)
                    pltpu.async_copy(mtab_hbm.at[idx_v], srows_v, sem).wait()
                    pltpu.sync_copy(
                        srows_v,
                        out.at[rows, pl.ds(COL_MULT + j * MULT_DIM, MULT_DIM)])

    return k(aidx, cidx, sidx, midx, atom_tab, ctab, stab, mtab, cmat)


def kernel(atom_idx, atom_charges, motif_features, shape_classes, mult_per_atom,
           atom_id_table, atom_charge_table, shape_id_table, atom_mult_table,
           W_motif, b_motif):
    sidx = (shape_classes.astype(jnp.int32) + 1).T.reshape(-1)
    midx = (mult_per_atom.astype(jnp.int32) + 1).T.reshape(-1)
    cidx = atom_charges.astype(jnp.int32) + 1

    w3 = jnp.zeros((NUM_JOINS * MOTIF_FEAT_SIZE, MOTIF_TOT), jnp.float32)
    for j in range(NUM_JOINS):
        w3 = w3.at[j * MOTIF_FEAT_SIZE:(j + 1) * MOTIF_FEAT_SIZE,
                   j * MOTIF_DIM:(j + 1) * MOTIF_DIM].set(W_motif)
    b3 = jnp.tile(b_motif, NUM_JOINS).reshape(1, MOTIF_TOT)

    cmat = _tc_motif(motif_features, w3, b3)
    return _sc_assemble(atom_idx.astype(jnp.int32), cidx, sidx, midx,
                        atom_id_table, atom_charge_table, shape_id_table,
                        atom_mult_table, cmat)


# speedup vs baseline: 1.2753x; 1.2753x over previous
import functools

import jax
import jax.numpy as jnp
from jax import lax
from jax.experimental import pallas as pl
from jax.experimental.pallas import tpu as pltpu
from jax.experimental.pallas import tpu_sc as plsc

N = 100000
ATOM_ID_DIM = 32
CHARGE_DIM = 8
SHAPE_ID_DIM = 16
MULT_DIM = 16
MOTIF_FEAT_SIZE = 48
MOTIF_DIM = 32
NUM_JOINS = 3
MOTIF_TOT = NUM_JOINS * MOTIF_DIM
OUT_DIM = 232
NSLOT = 8
GDIM = NSLOT * 16

SHAPE_OFS = 2 * N
MULT_OFS = SHAPE_OFS + 5001


CHUNK = 400
NUM_CHUNKS = N // CHUNK


def _sc_gather(idx8, cidx, mtab16, ctab16):
    info = plsc.get_sparse_core_info()
    nc, ns = info.num_cores, info.num_subcores
    nw = nc * ns
    chunks_per_w = -(-NUM_CHUNKS // nw)
    mesh = plsc.VectorSubcoreMesh(core_axis_name="c", subcore_axis_name="s")

    @functools.partial(
        pl.kernel,
        mesh=mesh,
        out_type=(
            jax.ShapeDtypeStruct((NSLOT * N, 16), jnp.float32),
            jax.ShapeDtypeStruct((N, 16), jnp.float32),
        ),
        scratch_types=[
            pltpu.VMEM((NSLOT * CHUNK,), jnp.int32),
            pltpu.VMEM((CHUNK,), jnp.int32),
            pltpu.VMEM((NSLOT * CHUNK, 16), jnp.float32),
            pltpu.VMEM((CHUNK, 16), jnp.float32),
            pltpu.SemaphoreType.DMA,
        ],
        compiler_params=pltpu.CompilerParams(use_tc_tiling_on_sc=False),
    )
    def k(idx8_hbm, cidx_hbm, mtab_hbm, ctab_hbm, g8_out, ch_out,
          idx_v, cidx_v, grows_v, crows_v, sem):
        wid = lax.axis_index("s") * nc + lax.axis_index("c")
        for c in range(chunks_per_w):
            cid = wid + nw * c

            @pl.when(cid < NUM_CHUNKS)
            def _():
                base = cid * CHUNK
                gr = pl.ds(base * NSLOT, NSLOT * CHUNK)
                pltpu.sync_copy(idx8_hbm.at[gr], idx_v)
                pltpu.async_copy(mtab_hbm.at[idx_v], grows_v, sem).wait()
                pltpu.sync_copy(grows_v, g8_out.at[gr])
                rows = pl.ds(base, CHUNK)
                pltpu.sync_copy(cidx_hbm.at[rows], cidx_v)
                pltpu.async_copy(ctab_hbm.at[cidx_v], crows_v, sem).wait()
                pltpu.sync_copy(crows_v, ch_out.at[rows])

    return k(idx8, cidx, mtab16, ctab16)


BR = 1000


def _tc_body(g_ref, mf_ref, ch_ref, pg_ref, pw_ref, pc_ref, b_ref, out_ref):
    acc = jnp.dot(g_ref[...], pg_ref[...], preferred_element_type=jnp.float32)
    acc += jnp.dot(mf_ref[...], pw_ref[...], preferred_element_type=jnp.float32)
    acc += jnp.dot(ch_ref[...], pc_ref[...], preferred_element_type=jnp.float32)
    out_ref[...] = acc + b_ref[...]


def _tc_assemble(g, mf, ch, pg, pw, pc, b232):
    return pl.pallas_call(
        _tc_body,
        grid=(N // BR,),
        in_specs=[
            pl.BlockSpec((BR, GDIM), lambda i: (i, 0)),
            pl.BlockSpec((BR, NUM_JOINS * MOTIF_FEAT_SIZE), lambda i: (i, 0)),
            pl.BlockSpec((BR, 16), lambda i: (i, 0)),
            pl.BlockSpec((GDIM, OUT_DIM), lambda i: (0, 0)),
            pl.BlockSpec((NUM_JOINS * MOTIF_FEAT_SIZE, OUT_DIM), lambda i: (0, 0)),
            pl.BlockSpec((16, OUT_DIM), lambda i: (0, 0)),
            pl.BlockSpec((1, OUT_DIM), lambda i: (0, 0)),
        ],
        out_specs=pl.BlockSpec((BR, OUT_DIM), lambda i: (i, 0)),
        out_shape=jax.ShapeDtypeStruct((N, OUT_DIM), jnp.float32),
        compiler_params=pltpu.CompilerParams(
            dimension_semantics=("arbitrary",),
        ),
    )(g, mf, ch, pg, pw, pc, b232)


def kernel(atom_idx, atom_charges, motif_features, shape_classes, mult_per_atom,
           atom_id_table, atom_charge_table, shape_id_table, atom_mult_table,
           W_motif, b_motif):
    f32 = jnp.float32
    mtab16 = jnp.concatenate([
        atom_id_table.reshape(2 * N, 16),
        shape_id_table,
        atom_mult_table,
    ], axis=0)
    ctab16 = jnp.zeros((3, 16), f32).at[:, :CHARGE_DIM].set(atom_charge_table)

    a2 = atom_idx.astype(jnp.int32) * 2
    idx8 = jnp.concatenate([
        a2[:, None], a2[:, None] + 1,
        shape_classes.astype(jnp.int32) + (1 + SHAPE_OFS),
        mult_per_atom.astype(jnp.int32) + (1 + MULT_OFS),
    ], axis=1).reshape(-1)
    cidx = atom_charges.astype(jnp.int32) + 1

    g8, ch = _sc_gather(idx8, cidx, mtab16, ctab16)
    g = g8.reshape(N, GDIM)

    pg = jnp.zeros((GDIM, OUT_DIM), f32)
    eye32 = jnp.eye(32, dtype=f32)
    pg = pg.at[0:32, 0:32].set(eye32)
    eye48 = jnp.eye(48, dtype=f32)
    pg = pg.at[32:80, 136:184].set(eye48)
    pg = pg.at[80:128, 184:232].set(eye48)

    pw = jnp.zeros((NUM_JOINS * MOTIF_FEAT_SIZE, OUT_DIM), f32)
    for j in range(NUM_JOINS):
        pw = pw.at[j * MOTIF_FEAT_SIZE:(j + 1) * MOTIF_FEAT_SIZE,
                   40 + j * MOTIF_DIM:40 + (j + 1) * MOTIF_DIM].set(W_motif)

    pc = jnp.zeros((16, OUT_DIM), f32)
    pc = pc.at[0:CHARGE_DIM, 32:40].set(jnp.eye(CHARGE_DIM, dtype=f32))

    b232 = jnp.zeros((1, OUT_DIM), f32)
    b232 = b232.at[0, 40:136].set(jnp.tile(b_motif, NUM_JOINS))

    return _tc_assemble(g, motif_features, ch, pg, pw, pc, b232)
